# baseline (speedup 1.0000x reference)
import functools

import jax
import jax.numpy as jnp
from jax import lax
from jax.experimental import pallas as pl
from jax.experimental.pallas import tpu as pltpu

N_STROKE = 100000
N_BREP = 50000
D = 32
_ROWS = 1000


def _prep_body(x_ref, w_ref, ws_ref, bs_ref, xs_ref, y_ref):
    xs = x_ref[...] * w_ref[...]
    xs_ref[...] = xs
    y_ref[...] = jnp.dot(xs, ws_ref[...], preferred_element_type=jnp.float32) + bs_ref[...]


def _prep(x_stroke, stroke_weights, Ws, bs):
    grid = (N_STROKE // _ROWS,)
    return pl.pallas_call(
        _prep_body,
        grid=grid,
        in_specs=[
            pl.BlockSpec((_ROWS, D), lambda i: (i, 0)),
            pl.BlockSpec((_ROWS, 1), lambda i: (i, 0)),
            pl.BlockSpec((D, D), lambda i: (0, 0)),
            pl.BlockSpec((1, D), lambda i: (0, 0)),
        ],
        out_specs=[
            pl.BlockSpec((_ROWS, D), lambda i: (i, 0)),
            pl.BlockSpec((_ROWS, D), lambda i: (i, 0)),
        ],
        out_shape=[
            jax.ShapeDtypeStruct((N_STROKE, D), jnp.float32),
            jax.ShapeDtypeStruct((N_STROKE, D), jnp.float32),
        ],
    )(x_stroke, stroke_weights, Ws, bs)


def _post_body(xs_ref, si_ref, st_ref, sr_ref, m_ref, ci_ref, ct_ref, cr_ref,
               wi_ref, bi_ref, wt_ref, bt_ref, wr_ref, br_ref,
               wl_ref, bl_ref, w1_ref, b1_ref, w2_ref, b2_ref, out_ref):
    xs = xs_ref[...]
    ci = ci_ref[...]
    ct = ct_ref[...]
    cr = cr_ref[...]
    a_i = jnp.dot(si_ref[...] / jnp.maximum(ci, 1.0), wi_ref[...],
                  preferred_element_type=jnp.float32) + (ci > 0) * bi_ref[...]
    a_t = jnp.dot(st_ref[...], wt_ref[...],
                  preferred_element_type=jnp.float32) + ct * bt_ref[...]
    a_r = jnp.dot(sr_ref[...] / jnp.maximum(cr, 1.0), wr_ref[...],
                  preferred_element_type=jnp.float32) + (cr > 0) * br_ref[...]
    m = m_ref[...]
    a_s = jnp.where(m == -jnp.inf, 0.0, m)
    out_stroke = jax.nn.relu(a_i + a_t + a_r + a_s) + xs
    feats = jnp.dot(out_stroke, wl_ref[...], preferred_element_type=jnp.float32) + bl_ref[...]
    h = jax.nn.relu(jnp.dot(feats, w1_ref[...], preferred_element_type=jnp.float32) + b1_ref[...])
    out_ref[...] = jax.nn.sigmoid(
        jnp.dot(h, w2_ref[...], preferred_element_type=jnp.float32) + b2_ref[...])


def _post(xs, S_i, S_t, S_r, M, c_i, c_t, c_r,
          Wi, bi, Wt, bt, Wr, br, Wl, bl, W1, b1, W2, b2):
    grid = (N_STROKE // _ROWS,)
    row = lambda i: (i, 0)
    full = lambda shape: pl.BlockSpec(shape, lambda i: (0, 0))
    return pl.pallas_call(
        _post_body,
        grid=grid,
        in_specs=[
            pl.BlockSpec((_ROWS, D), row),
            pl.BlockSpec((_ROWS, D), row),
            pl.BlockSpec((_ROWS, D), row),
            pl.BlockSpec((_ROWS, D), row),
            pl.BlockSpec((_ROWS, D), row),
            pl.BlockSpec((_ROWS, 1), row),
            pl.BlockSpec((_ROWS, 1), row),
            pl.BlockSpec((_ROWS, 1), row),
            full((D, D)), full((1, D)),
            full((D, D)), full((1, D)),
            full((D, D)), full((1, D)),
            full((D, 64)), full((1, 64)),
            full((64, 128)), full((1, 128)),
            full((128, 1)), full((1, 1)),
        ],
        out_specs=pl.BlockSpec((_ROWS, 1), row),
        out_shape=jax.ShapeDtypeStruct((N_STROKE, 1), jnp.float32),
    )(xs, S_i, S_t, S_r, M, c_i, c_t, c_r,
      Wi, bi, Wt, bt, Wr, br, Wl, bl, W1, b1, W2, b2)


def kernel(x_stroke, x_brep, stroke_weights,
           ei_intersects, ei_temp_previous, ei_represented_by,
           ei_brepcoplanar, ei_strokecoplanar,
           W_intersects, b_intersects, W_temp_previous, b_temp_previous,
           W_represented_by, b_represented_by, W_brepcoplanar, b_brepcoplanar,
           W_strokecoplanar, b_strokecoplanar,
           W_local, b_local, W_dec1, b_dec1, W_dec2, b_dec2):
    xs, y = _prep(x_stroke, stroke_weights, W_strokecoplanar,
                  b_strokecoplanar.reshape(1, D))

    def seg_sum(table, ei, n):
        s = jax.ops.segment_sum(table[ei[0]], ei[1], num_segments=n)
        c = jax.ops.segment_sum(jnp.ones((ei.shape[1],), jnp.float32), ei[1],
                                num_segments=n)
        return s, c.reshape(n, 1)

    S_i, c_i = seg_sum(xs, ei_intersects, N_STROKE)
    S_t, c_t = seg_sum(xs, ei_temp_previous, N_STROKE)
    S_r, c_r = seg_sum(x_brep, ei_represented_by, N_STROKE)
    M = jax.ops.segment_max(y[ei_strokecoplanar[0]], ei_strokecoplanar[1],
                            num_segments=N_STROKE)

    return _post(xs, S_i, S_t, S_r, M, c_i, c_t, c_r,
                 W_intersects, b_intersects.reshape(1, D),
                 W_temp_previous, b_temp_previous.reshape(1, D),
                 W_represented_by, b_represented_by.reshape(1, D),
                 W_local, b_local.reshape(1, 64),
                 W_dec1, b_dec1.reshape(1, 128),
                 W_dec2, b_dec2.reshape(1, 1))


# speedup vs baseline: 3.1732x; 3.1732x over previous
import functools

import jax
import jax.numpy as jnp
from jax import lax
from jax.experimental import pallas as pl
from jax.experimental.pallas import tpu as pltpu
from jax.experimental.pallas import tpu_sc as plsc

N_STROKE = 100000
N_BREP = 50000
D = 32
_ROWS = 1024

_NW = 32
_E = 1600000
_CW = 2000
_IOTA = lambda: lax.iota(jnp.int32, 16)


def _count_body(di_hbm, dt_hbm, dr_hbm, oi_hbm, ot_hbm, or_hbm, dwin, hist):
    wid = lax.axis_index("s") * 2 + lax.axis_index("c")
    ew = _E // _NW
    ones = jnp.ones((16,), jnp.int32)

    for d_hbm, o_hbm in ((di_hbm, oi_hbm), (dt_hbm, ot_hbm), (dr_hbm, or_hbm)):
        def zero(j, _):
            hist[pl.ds(j * 16, 16)] = jnp.zeros((16,), jnp.int32)
            return 0
        lax.fori_loop(0, N_STROKE // 16, zero, 0, unroll=False)

        def win(it, _):
            pltpu.sync_copy(d_hbm.at[pl.ds(wid * ew + it * _CW, _CW)], dwin)

            def chunk(k, _):
                d16 = dwin[pl.ds(k * 16, 16)]
                plsc.addupdate_scatter(hist, [d16], ones)
                return 0
            lax.fori_loop(0, _CW // 16, chunk, 0, unroll=False)
            return 0
        lax.fori_loop(0, ew // _CW, win, 0, unroll=False)
        pltpu.sync_copy(hist, o_hbm.at[wid])


def _counts(dst_i, dst_t, dst_r):
    mesh = plsc.VectorSubcoreMesh(core_axis_name="c", subcore_axis_name="s")
    out = jax.ShapeDtypeStruct((_NW, N_STROKE), jnp.int32)
    fn = pl.kernel(
        _count_body,
        out_type=[out, out, out],
        mesh=mesh,
        compiler_params=pltpu.CompilerParams(needs_layout_passes=False),
        scratch_types=[
            pltpu.VMEM((_CW,), jnp.int32),
            pltpu.VMEM((N_STROKE,), jnp.int32),
        ],
    )
    return fn(dst_i, dst_t, dst_r)


_W = 2000
_NSUB = 16
_QTR = N_STROKE // 4
_ACC = _QTR + 208
_ACC16 = 25216
_EWS = _E // _NSUB


def _sum_body(tab_s, tab_b, src_i, dst_i, src_t, dst_t, src_r, dst_r,
              zeros_hbm, out_i, out_t, out_r,
              srcw, dstw, srcp, dstlp, dstl0, dstl1, dstl2, dstl3,
              dstl4, dstl5, dstl6, dstl7, rows, acc, gsem, ssem):
    core = lax.axis_index("c")
    sub = lax.axis_index("s")
    iota = lax.iota(jnp.int32, 16)
    dstl = (dstl0, dstl1, dstl2, dstl3, dstl4, dstl5, dstl6, dstl7)

    for tab, src_hbm, dst_hbm, out_hbm in (
            (tab_s, src_i, dst_i, out_i),
            (tab_s, src_t, dst_t, out_t),
            (tab_b, src_r, dst_r, out_r)):
      for pss in range(2):
        lo = (2 * pss + core) * _QTR
        pltpu.sync_copy(zeros_hbm.at[pl.ds(sub * 1576, 1576)],
                        acc.at[pl.ds(sub * 1576, 1576)])
        plsc.subcore_barrier()

        def win(it, off):
            wb = sub * _EWS + it * _W
            pltpu.sync_copy(src_hbm.at[pl.ds(wb, _W)], srcw)
            pltpu.sync_copy(dst_hbm.at[pl.ds(wb, _W)], dstw)

            def chunk(k, off):
                d = dstw[pl.ds(k * 16, 16)]
                s16 = srcw[pl.ds(k * 16, 16)]
                l = d - lo
                m = (l >= 0) & (l < _QTR)
                cnt = jnp.sum(m.astype(jnp.int32))
                plsc.store_compressed(srcp.at[pl.ds(off, 16)], s16, mask=m)
                plsc.store_compressed(dstlp.at[pl.ds(off, 16)], l, mask=m)
                return off + cnt
            off = lax.fori_loop(0, _W // 16, chunk, off, unroll=False)

            nfull = off // 128

            def batch(b, _):
                bstart = b * 8
                nb = nfull - bstart

                for j in range(8):
                    @pl.when(j < nb)
                    def _(j=j):
                        c = bstart + j

                        def mv(k, _):
                            dstl[j][pl.ds(k * 16, 16)] = (
                                dstlp[pl.ds(c * 128 + k * 16, 16)])
                            return 0
                        lax.fori_loop(0, 8, mv, 0, unroll=True)
                        pltpu.async_copy(tab.at[srcp.at[pl.ds(c * 128, 128)]],
                                         rows.at[pl.ds(j * 128, 128)], gsem)

                for j in range(8):
                    @pl.when(j < nb)
                    def _(j=j):
                        pltpu.make_async_copy(
                            tab.at[srcp.at[pl.ds((bstart + j) * 128, 128)]],
                            rows.at[pl.ds(j * 128, 128)], gsem).wait()

                for j in range(8):
                    @pl.when(j < nb)
                    def _(j=j):
                        pltpu.async_copy(rows.at[pl.ds(j * 128, 128)],
                                         acc.at[dstl[j]], ssem, add=True)

                for j in range(8):
                    @pl.when(j < nb)
                    def _(j=j):
                        pltpu.make_async_copy(rows.at[pl.ds(j * 128, 128)],
                                              acc.at[dstl[j]], ssem).wait()
                return 0
            lax.fori_loop(0, (nfull + 7) // 8, batch, 0, unroll=False)

            rem = off - nfull * 128

            def shift(k, _):
                srcp[pl.ds(k * 16, 16)] = srcp[pl.ds(nfull * 128 + k * 16, 16)]
                dstlp[pl.ds(k * 16, 16)] = dstlp[pl.ds(nfull * 128 + k * 16, 16)]
                return 0
            lax.fori_loop(0, 8, shift, 0, unroll=False)
            return rem
        off_end = lax.fori_loop(0, _EWS // _W, win, 0, unroll=False)

        @pl.when(off_end > 0)
        def _():
            def pad(k, _):
                srcp[pl.ds(off_end + k * 16, 16)] = iota + k * 16
                dstlp[pl.ds(off_end + k * 16, 16)] = _QTR + 48 + iota + k * 16
                return 0
            lax.fori_loop(0, 8, pad, 0, unroll=False)

            def mv(k, _):
                dstl0[pl.ds(k * 16, 16)] = dstlp[pl.ds(k * 16, 16)]
                return 0
            lax.fori_loop(0, 8, mv, 0, unroll=False)
            pltpu.async_copy(tab.at[srcp.at[pl.ds(0, 128)]],
                             rows.at[pl.ds(0, 128)], gsem).wait()
            pltpu.sync_copy(rows.at[pl.ds(0, 128)], acc.at[dstl0],
                            add=True)

        plsc.subcore_barrier()
        osz = jnp.where(sub == 15, 1240, 1584)
        pltpu.sync_copy(acc.at[pl.ds(sub * 1584, osz)],
                        out_hbm.at[pl.ds(lo + sub * 1584, osz)])
        plsc.subcore_barrier()


def _sums(xs, xb, ei_i, ei_t, ei_r):
    mesh = plsc.VectorSubcoreMesh(core_axis_name="c", subcore_axis_name="s")
    out = jax.ShapeDtypeStruct((N_STROKE, D), jnp.float32)
    fn = pl.kernel(
        _sum_body,
        out_type=[out, out, out],
        mesh=mesh,
        compiler_params=pltpu.CompilerParams(needs_layout_passes=False,
                                             use_tc_tiling_on_sc=False),
        scratch_types=[
            pltpu.VMEM((_W,), jnp.int32),
            pltpu.VMEM((_W,), jnp.int32),
            pltpu.VMEM((4224,), jnp.int32),
            pltpu.VMEM((4224,), jnp.int32),
            pltpu.VMEM((128,), jnp.int32),
            pltpu.VMEM((128,), jnp.int32),
            pltpu.VMEM((128,), jnp.int32),
            pltpu.VMEM((128,), jnp.int32),
            pltpu.VMEM((128,), jnp.int32),
            pltpu.VMEM((128,), jnp.int32),
            pltpu.VMEM((128,), jnp.int32),
            pltpu.VMEM((128,), jnp.int32),
            pltpu.VMEM((1024, D), jnp.float32),
            pltpu.VMEM_SHARED((_ACC16, D), jnp.float32),
            pltpu.SemaphoreType.DMA,
            pltpu.SemaphoreType.DMA,
        ],
    )
    zeros = jnp.zeros((_ACC16, D), jnp.float32)
    return fn(xs, xb, ei_i[0], ei_i[1], ei_t[0], ei_t[1], ei_r[0], ei_r[1],
              zeros)


def _prep_body(x_ref, w_ref, ws_ref, bs_ref, xs_ref, y_ref):
    xs = x_ref[...] * w_ref[...]
    xs_ref[...] = xs
    y_ref[...] = jnp.dot(xs, ws_ref[...], preferred_element_type=jnp.float32) + bs_ref[...]


def _prep(x_stroke, stroke_weights, Ws, bs):
    grid = (pl.cdiv(N_STROKE, _ROWS),)
    return pl.pallas_call(
        _prep_body,
        grid=grid,
        in_specs=[
            pl.BlockSpec((_ROWS, D), lambda i: (i, 0)),
            pl.BlockSpec((_ROWS, 1), lambda i: (i, 0)),
            pl.BlockSpec((D, D), lambda i: (0, 0)),
            pl.BlockSpec((1, D), lambda i: (0, 0)),
        ],
        out_specs=[
            pl.BlockSpec((_ROWS, D), lambda i: (i, 0)),
            pl.BlockSpec((_ROWS, D), lambda i: (i, 0)),
        ],
        out_shape=[
            jax.ShapeDtypeStruct((N_STROKE, D), jnp.float32),
            jax.ShapeDtypeStruct((N_STROKE, D), jnp.float32),
        ],
    )(x_stroke, stroke_weights, Ws, bs)


def _post_body(xs_ref, si_ref, st_ref, sr_ref, m_ref, ci_ref, ct_ref, cr_ref,
               wi_ref, bi_ref, wt_ref, bt_ref, wr_ref, br_ref,
               wl_ref, bl_ref, w1_ref, b1_ref, w2_ref, b2_ref, out_ref):
    xs = xs_ref[...]
    ci = jnp.sum(ci_ref[...], axis=0).astype(jnp.float32)[:, None]
    ct = jnp.sum(ct_ref[...], axis=0).astype(jnp.float32)[:, None]
    cr = jnp.sum(cr_ref[...], axis=0).astype(jnp.float32)[:, None]
    a_i = jnp.dot(si_ref[...] / jnp.maximum(ci, 1.0), wi_ref[...],
                  preferred_element_type=jnp.float32) + (ci > 0) * bi_ref[...]
    a_t = jnp.dot(st_ref[...], wt_ref[...],
                  preferred_element_type=jnp.float32) + ct * bt_ref[...]
    a_r = jnp.dot(sr_ref[...] / jnp.maximum(cr, 1.0), wr_ref[...],
                  preferred_element_type=jnp.float32) + (cr > 0) * br_ref[...]
    m = m_ref[...]
    a_s = jnp.where(m == -jnp.inf, 0.0, m)
    out_stroke = jax.nn.relu(a_i + a_t + a_r + a_s) + xs
    feats = jnp.dot(out_stroke, wl_ref[...], preferred_element_type=jnp.float32) + bl_ref[...]
    h = jax.nn.relu(jnp.dot(feats, w1_ref[...], preferred_element_type=jnp.float32) + b1_ref[...])
    out_ref[...] = jax.nn.sigmoid(
        jnp.dot(h, w2_ref[...], preferred_element_type=jnp.float32) + b2_ref[...])


def _post(xs, S_i, S_t, S_r, M, c_i, c_t, c_r,
          Wi, bi, Wt, bt, Wr, br, Wl, bl, W1, b1, W2, b2):
    grid = (pl.cdiv(N_STROKE, _ROWS),)
    row = lambda i: (i, 0)
    full = lambda shape: pl.BlockSpec(shape, lambda i: (0, 0))
    return pl.pallas_call(
        _post_body,
        grid=grid,
        in_specs=[
            pl.BlockSpec((_ROWS, D), row),
            pl.BlockSpec((_ROWS, D), row),
            pl.BlockSpec((_ROWS, D), row),
            pl.BlockSpec((_ROWS, D), row),
            pl.BlockSpec((_ROWS, D), row),
            pl.BlockSpec((_NW, _ROWS), lambda i: (0, i)),
            pl.BlockSpec((_NW, _ROWS), lambda i: (0, i)),
            pl.BlockSpec((_NW, _ROWS), lambda i: (0, i)),
            full((D, D)), full((1, D)),
            full((D, D)), full((1, D)),
            full((D, D)), full((1, D)),
            full((D, 64)), full((1, 64)),
            full((64, 128)), full((1, 128)),
            full((128, 1)), full((1, 1)),
        ],
        out_specs=pl.BlockSpec((_ROWS, 1), row),
        out_shape=jax.ShapeDtypeStruct((N_STROKE, 1), jnp.float32),
    )(xs, S_i, S_t, S_r, M, c_i, c_t, c_r,
      Wi, bi, Wt, bt, Wr, br, Wl, bl, W1, b1, W2, b2)


def kernel(x_stroke, x_brep, stroke_weights,
           ei_intersects, ei_temp_previous, ei_represented_by,
           ei_brepcoplanar, ei_strokecoplanar,
           W_intersects, b_intersects, W_temp_previous, b_temp_previous,
           W_represented_by, b_represented_by, W_brepcoplanar, b_brepcoplanar,
           W_strokecoplanar, b_strokecoplanar,
           W_local, b_local, W_dec1, b_dec1, W_dec2, b_dec2):
    xs, y = _prep(x_stroke, stroke_weights, W_strokecoplanar,
                  b_strokecoplanar.reshape(1, D))

    S_i, S_t, S_r = _sums(xs, x_brep, ei_intersects, ei_temp_previous,
                          ei_represented_by)
    c_i, c_t, c_r = _counts(ei_intersects[1], ei_temp_previous[1],
                            ei_represented_by[1])
    M = jax.ops.segment_max(y[ei_strokecoplanar[0]], ei_strokecoplanar[1],
                            num_segments=N_STROKE)

    return _post(xs, S_i, S_t, S_r, M, c_i, c_t, c_r,
                 W_intersects, b_intersects.reshape(1, D),
                 W_temp_previous, b_temp_previous.reshape(1, D),
                 W_represented_by, b_represented_by.reshape(1, D),
                 W_local, b_local.reshape(1, 64),
                 W_dec1, b_dec1.reshape(1, 128),
                 W_dec2, b_dec2.reshape(1, 1))
